# baseline (speedup 1.0000x reference)
import functools

import jax
import jax.numpy as jnp
from jax.experimental import pallas as pl
from jax.experimental.pallas import tpu as pltpu

N = 10000
E = 160000
HEADS = 8


def _mm_body(a_ref, w_ref, b_ref, o_ref):
    o_ref[...] = (
        jnp.dot(a_ref[...], w_ref[...], preferred_element_type=jnp.float32)
        + b_ref[...]
    )


def _matmul_bias(a, w, b, block_rows=1000):
    n, k = a.shape
    m = w.shape[1]
    assert n % block_rows == 0
    grid = (n // block_rows,)
    return pl.pallas_call(
        _mm_body,
        grid=grid,
        in_specs=[
            pl.BlockSpec((block_rows, k), lambda i: (i, 0)),
            pl.BlockSpec((k, m), lambda i: (0, 0)),
            pl.BlockSpec((m,), lambda i: (0,)),
        ],
        out_specs=pl.BlockSpec((block_rows, m), lambda i: (i, 0)),
        out_shape=jax.ShapeDtypeStruct((n, m), jnp.float32),
    )(a, w, b)


def _gatv2_layer(x, src, dst, Wl, bl, Wr, br, att, bias, heads, out_ch):
    n = x.shape[0]
    x_l = _matmul_bias(x, Wl, bl).reshape(n, heads, out_ch)
    x_r = _matmul_bias(x, Wr, br).reshape(n, heads, out_ch)
    e = x_l[src] + x_r[dst]
    e = jnp.where(e > 0, e, 0.2 * e)
    score = jnp.sum(e * att[None, :, :], axis=-1)
    smax = jax.ops.segment_max(score, dst, num_segments=n)
    score = jnp.exp(score - smax[dst])
    denom = jax.ops.segment_sum(score, dst, num_segments=n)
    alpha = score / (denom[dst] + 1e-16)
    msg = alpha[:, :, None] * x_l[src]
    out = jax.ops.segment_sum(msg, dst, num_segments=n)
    return out.reshape(n, heads * out_ch) + bias


def kernel(x, edge_index, Wl1, bl1, Wr1, br1, att1, bias1,
           Wl2, bl2, Wr2, br2, att2, bias2, Wlin, blin):
    n = x.shape[0]
    loops = jnp.arange(n, dtype=edge_index.dtype)
    src = jnp.concatenate([edge_index[0], loops])
    dst = jnp.concatenate([edge_index[1], loops])
    h = _gatv2_layer(x, src, dst, Wl1, bl1, Wr1, br1, att1, bias1, HEADS, 64)
    h = jax.nn.relu(h)
    h = _gatv2_layer(h, src, dst, Wl2, bl2, Wr2, br2, att2, bias2, HEADS, 128)
    return _matmul_bias(h, Wlin, blin)


# speedup vs baseline: 1.7915x; 1.7915x over previous
import functools

import jax
import jax.numpy as jnp
from jax import lax
from jax.experimental import pallas as pl
from jax.experimental.pallas import tpu as pltpu
from jax.experimental.pallas import tpu_sc as plsc

N = 10000
E = 160000
HEADS = 8
NC = 2
NS = 16
NW = NC * NS
LANES = 16
KC = 32
EP = 167 * NW * KC
NPT = N // NS

_SC_MESH = plsc.VectorSubcoreMesh(core_axis_name="c", subcore_axis_name="s")
_SC_PARAMS = pltpu.CompilerParams(use_tc_tiling_on_sc=False,
                                  needs_layout_passes=False)


def _mm2_body(x_ref, wl_ref, bl_ref, wr_ref, br_ref, ol_ref, or_ref):
    x = x_ref[...]
    ol_ref[...] = (
        jnp.dot(x, wl_ref[...], preferred_element_type=jnp.float32) + bl_ref[...]
    )
    or_ref[...] = (
        jnp.dot(x, wr_ref[...], preferred_element_type=jnp.float32) + br_ref[...]
    )


def _proj_pair(x, wl, bl, wr, br, block_rows=1000):
    n, k = x.shape
    m = wl.shape[1]
    grid = (n // block_rows,)
    return pl.pallas_call(
        _mm2_body,
        grid=grid,
        in_specs=[
            pl.BlockSpec((block_rows, k), lambda i: (i, 0)),
            pl.BlockSpec((k, m), lambda i: (0, 0)),
            pl.BlockSpec((m,), lambda i: (0,)),
            pl.BlockSpec((k, m), lambda i: (0, 0)),
            pl.BlockSpec((m,), lambda i: (0,)),
        ],
        out_specs=[
            pl.BlockSpec((block_rows, m), lambda i: (i, 0)),
            pl.BlockSpec((block_rows, m), lambda i: (i, 0)),
        ],
        out_shape=[
            jax.ShapeDtypeStruct((n, m), jnp.float32),
            jax.ShapeDtypeStruct((n, m), jnp.float32),
        ],
    )(x, wl, bl, wr, br)


def _make_score_kernel(d, c):
    epw = EP // NW
    nchunks = epw // KC

    @functools.partial(
        pl.kernel,
        out_type=[
            jax.ShapeDtypeStruct((EP, HEADS), jnp.float32),
            jax.ShapeDtypeStruct((NC, N, HEADS), jnp.float32),
        ],
        mesh=_SC_MESH,
        scratch_types=[
            pltpu.VMEM((KC,), jnp.int32),
            pltpu.VMEM((KC,), jnp.int32),
            pltpu.VMEM((KC, d), jnp.float32),
            pltpu.VMEM((KC, d), jnp.float32),
            pltpu.VMEM((KC, HEADS), jnp.float32),
            pltpu.VMEM((HEADS, c), jnp.float32),
            pltpu.VMEM_SHARED((N, HEADS), jnp.float32),
            pltpu.SemaphoreType.DMA,
            pltpu.SemaphoreType.DMA,
        ],
        compiler_params=_SC_PARAMS,
    )
    def score_kernel(src_hbm, dst_hbm, xl_hbm, xr_hbm, att_hbm, zd_hbm,
                     w_hbm, den_hbm,
                     src_v, dst_v, xlr, xrr, wbuf, att_v, den_sp, sem1, sem2):
        cid = lax.axis_index("c")
        sid = lax.axis_index("s")
        wid = sid * NC + cid

        pltpu.sync_copy(att_hbm, att_v)

        @pl.when(sid == 0)
        def _():
            pltpu.sync_copy(zd_hbm, den_sp)

        plsc.subcore_barrier()

        base0 = wid * epw

        @pl.loop(0, nchunks)
        def _chunk(ch):
            base = base0 + ch * KC
            pltpu.sync_copy(src_hbm.at[pl.ds(base, KC)], src_v)
            pltpu.sync_copy(dst_hbm.at[pl.ds(base, KC)], dst_v)
            g1 = pltpu.async_copy(xl_hbm.at[src_v], xlr, sem1)
            g2 = pltpu.async_copy(xr_hbm.at[dst_v], xrr, sem2)
            g1.wait()
            g2.wait()
            for grp in range(KC // LANES):
                erow = lax.iota(jnp.int32, LANES) + grp * LANES
                eglob = erow + base
                inb = eglob < (E + N)
                for h in range(HEADS):
                    hrow = jnp.full((LANES,), h, jnp.int32)
                    s0 = jnp.zeros((LANES,), jnp.float32)

                    @pl.loop(0, c, init_carry=s0, unroll=4)
                    def _score(cc, s):
                        colv = jnp.full((LANES,), h * c, jnp.int32) + cc
                        a = plsc.load_gather(xlr, [erow, colv])
                        b = plsc.load_gather(xrr, [erow, colv])
                        t = a + b
                        t = jnp.where(t > 0, t, 0.2 * t)
                        av = plsc.load_gather(
                            att_v, [hrow, jnp.full((LANES,), cc, jnp.int32)])
                        return s + av * t

                    w = jnp.where(inb, jnp.exp(_score), 0.0)
                    plsc.store_scatter(wbuf, [erow, hrow], w)
            pltpu.sync_copy(wbuf, w_hbm.at[pl.ds(base, KC)])
            pltpu.sync_copy(wbuf, den_sp.at[dst_v], add=True)

        plsc.subcore_barrier()

        @pl.when(sid == 0)
        def _():
            pltpu.sync_copy(den_sp, den_hbm.at[cid])

    return score_kernel


def _make_message_kernel(d, c):
    hpc = HEADS // NC
    epw = EP // NS
    nchunks = epw // KC

    @functools.partial(
        pl.kernel,
        out_type=jax.ShapeDtypeStruct((HEADS, N, c), jnp.float32),
        mesh=_SC_MESH,
        scratch_types=[
            pltpu.VMEM((KC,), jnp.int32),
            pltpu.VMEM((KC,), jnp.int32),
            pltpu.VMEM((KC,), jnp.int32),
            pltpu.VMEM((KC, c), jnp.float32),
            pltpu.VMEM((KC, HEADS), jnp.float32),
            pltpu.VMEM_SHARED((N, c), jnp.float32),
            pltpu.SemaphoreType.DMA,
        ],
        compiler_params=_SC_PARAMS,
    )
    def message_kernel(src_hbm, dst_hbm, xlhc_hbm, w_hbm, zn_hbm,
                       num_hbm,
                       src_v, dst_v, ridx_v, rows, wall, num_sp, sem1):
        cid = lax.axis_index("c")
        sid = lax.axis_index("s")
        base0 = sid * epw

        for p in range(hpc):
            hglob_s = cid * hpc + p
            pltpu.sync_copy(zn_hbm.at[pl.ds(sid * NPT, NPT)],
                            num_sp.at[pl.ds(sid * NPT, NPT)])
            plsc.subcore_barrier()

            @pl.loop(0, nchunks)
            def _chunk(ch):
                base = base0 + ch * KC
                pltpu.sync_copy(src_hbm.at[pl.ds(base, KC)], src_v)
                pltpu.sync_copy(dst_hbm.at[pl.ds(base, KC)], dst_v)
                pltpu.sync_copy(w_hbm.at[pl.ds(base, KC)], wall)
                for grp in range(KC // LANES):
                    sv = src_v[pl.ds(grp * LANES, LANES)]
                    ridx_v[pl.ds(grp * LANES, LANES)] = sv * HEADS + hglob_s
                g1 = pltpu.async_copy(xlhc_hbm.at[ridx_v], rows, sem1)
                g1.wait()
                for grp in range(KC // LANES):
                    erow = lax.iota(jnp.int32, LANES) + grp * LANES
                    hrow = jnp.full((LANES,), 0, jnp.int32) + hglob_s
                    wv = plsc.load_gather(wall, [erow, hrow])

                    @pl.loop(0, c, unroll=4)
                    def _scale(cc):
                        colv = jnp.full((LANES,), 0, jnp.int32) + cc
                        m = plsc.load_gather(rows, [erow, colv])
                        plsc.store_scatter(rows, [erow, colv], m * wv)

                pltpu.sync_copy(rows, num_sp.at[dst_v], add=True)

            plsc.subcore_barrier()
            pltpu.sync_copy(num_sp.at[pl.ds(sid * NPT, NPT)],
                            num_hbm.at[hglob_s, pl.ds(sid * NPT, NPT)])
            plsc.subcore_barrier()

    return message_kernel


def _combine_mm_body(num_ref, den_ref, bias_ref, *args, c, relu, n_out):
    w_refs = args[:n_out]
    b_refs = args[n_out:2 * n_out]
    o_refs = args[2 * n_out:]
    den = den_ref[0] + den_ref[1]
    accs = [jnp.zeros(o.shape, jnp.float32) for o in o_refs]
    for h in range(HEADS):
        seg = (num_ref[h] / (den[:, h:h + 1] + 1e-16)
               + bias_ref[0, h * c:(h + 1) * c])
        if relu:
            seg = jnp.maximum(seg, 0.0)
        for j, w_ref in enumerate(w_refs):
            accs[j] += jnp.dot(seg, w_ref[h * c:(h + 1) * c, :],
                               preferred_element_type=jnp.float32)
    for j, o_ref in enumerate(o_refs):
        o_ref[...] = accs[j] + b_refs[j][...]


def _combine_mm(num, den, bias, ws, bs, *, relu, block_rows=1000):
    c = num.shape[2]
    n_out = len(ws)
    grid = (N // block_rows,)
    body = functools.partial(_combine_mm_body, c=c, relu=relu, n_out=n_out)
    in_specs = [
        pl.BlockSpec((HEADS, block_rows, c), lambda i: (0, i, 0)),
        pl.BlockSpec((NC, block_rows, HEADS), lambda i: (0, i, 0)),
        pl.BlockSpec((1, HEADS * c), lambda i: (0, 0)),
    ]
    for w in ws:
        in_specs.append(pl.BlockSpec(w.shape, lambda i: (0, 0)))
    for b in bs:
        in_specs.append(pl.BlockSpec(b.shape, lambda i: (0,)))
    outs = pl.pallas_call(
        body,
        grid=grid,
        in_specs=in_specs,
        out_specs=[pl.BlockSpec((block_rows, w.shape[1]), lambda i: (i, 0))
                   for w in ws],
        out_shape=[jax.ShapeDtypeStruct((N, w.shape[1]), jnp.float32)
                   for w in ws],
    )(num, den, bias.reshape(1, -1), *ws, *bs)
    return outs


def kernel(x, edge_index, Wl1, bl1, Wr1, br1, att1, bias1,
           Wl2, bl2, Wr2, br2, att2, bias2, Wlin, blin):
    loops = jnp.arange(N, dtype=edge_index.dtype)
    pad = jnp.zeros((EP - E - N,), dtype=edge_index.dtype)
    src = jnp.concatenate([edge_index[0], loops, pad])
    dst = jnp.concatenate([edge_index[1], loops, pad])

    xl1, xr1 = _proj_pair(x, Wl1, bl1, Wr1, br1)
    zd = jnp.zeros((N, HEADS), jnp.float32)
    w1, den1 = _make_score_kernel(512, 64)(src, dst, xl1, xr1, att1, zd)
    zn1 = jnp.zeros((N, 64), jnp.float32)
    num1 = _make_message_kernel(512, 64)(
        src, dst, xl1.reshape(N * HEADS, 64), w1, zn1)

    xl2, xr2 = _combine_mm(num1, den1, bias1, [Wl2, Wr2], [bl2, br2],
                           relu=True)

    w2, den2 = _make_score_kernel(1024, 128)(src, dst, xl2, xr2, att2, zd)
    zn2 = jnp.zeros((N, 128), jnp.float32)
    num2 = _make_message_kernel(1024, 128)(
        src, dst, xl2.reshape(N * HEADS, 128), w2, zn2)

    (out,) = _combine_mm(num2, den2, bias2, [Wlin], [blin], relu=False)
    return out


# speedup vs baseline: 2.4268x; 1.3546x over previous
import functools

import jax
import jax.numpy as jnp
from jax import lax
from jax.experimental import pallas as pl
from jax.experimental.pallas import tpu as pltpu
from jax.experimental.pallas import tpu_sc as plsc

N = 10000
E = 160000
HEADS = 8
NC = 2
NS = 16
NW = NC * NS
LANES = 16
KC = 32
EP = 167 * NW * KC
NPT = N // NS

_SC_MESH = plsc.VectorSubcoreMesh(core_axis_name="c", subcore_axis_name="s")
_SC_PARAMS = pltpu.CompilerParams(use_tc_tiling_on_sc=False,
                                  needs_layout_passes=False)


def _mm2_body(x_ref, wl_ref, bl_ref, wr_ref, br_ref, ol_ref, or_ref):
    x = x_ref[...]
    ol_ref[...] = (
        jnp.dot(x, wl_ref[...], preferred_element_type=jnp.float32) + bl_ref[...]
    )
    or_ref[...] = (
        jnp.dot(x, wr_ref[...], preferred_element_type=jnp.float32) + br_ref[...]
    )


def _proj_pair(x, wl, bl, wr, br, block_rows=1000):
    n, k = x.shape
    m = wl.shape[1]
    grid = (n // block_rows,)
    return pl.pallas_call(
        _mm2_body,
        grid=grid,
        in_specs=[
            pl.BlockSpec((block_rows, k), lambda i: (i, 0)),
            pl.BlockSpec((k, m), lambda i: (0, 0)),
            pl.BlockSpec((m,), lambda i: (0,)),
            pl.BlockSpec((k, m), lambda i: (0, 0)),
            pl.BlockSpec((m,), lambda i: (0,)),
        ],
        out_specs=[
            pl.BlockSpec((block_rows, m), lambda i: (i, 0)),
            pl.BlockSpec((block_rows, m), lambda i: (i, 0)),
        ],
        out_shape=[
            jax.ShapeDtypeStruct((n, m), jnp.float32),
            jax.ShapeDtypeStruct((n, m), jnp.float32),
        ],
    )(x, wl, bl, wr, br)


def _make_score_kernel(d, c, kc):
    epw = EP // NW
    nchunks = epw // kc
    npairs = nchunks // 2
    tail = nchunks - 2 * npairs

    @functools.partial(
        pl.kernel,
        out_type=[
            jax.ShapeDtypeStruct((EP, HEADS), jnp.float32),
            jax.ShapeDtypeStruct((NC, N, HEADS), jnp.float32),
        ],
        mesh=_SC_MESH,
        scratch_types=[
            pltpu.VMEM((2, kc), jnp.int32),
            pltpu.VMEM((2, kc), jnp.int32),
            pltpu.VMEM((2, kc), jnp.int32),
            pltpu.VMEM((2, kc, d), jnp.float32),
            pltpu.VMEM((2, kc, d), jnp.float32),
            pltpu.VMEM((2, kc, HEADS), jnp.float32),
            pltpu.VMEM((HEADS, c), jnp.float32),
            pltpu.VMEM_SHARED((N, HEADS), jnp.float32),
            pltpu.SemaphoreType.DMA,
            pltpu.SemaphoreType.DMA,
            pltpu.SemaphoreType.DMA,
            pltpu.SemaphoreType.DMA,
            pltpu.SemaphoreType.DMA,
            pltpu.SemaphoreType.DMA,
        ],
        compiler_params=_SC_PARAMS,
    )
    def score_kernel(src_hbm, dst_hbm, xl_hbm, xr_hbm, att_hbm, zd_hbm,
                     w_hbm, den_hbm,
                     src_v, dst_v, dsto_v, xlr, xrr, wbuf, att_v, den_sp,
                     semA0, semA1, semG0, semG1, semO0, semO1):
        cid = lax.axis_index("c")
        sid = lax.axis_index("s")
        wid = sid * NC + cid
        semA = [semA0, semA1]
        semG = [semG0, semG1]
        semO = [semO0, semO1]

        pltpu.sync_copy(att_hbm, att_v)

        @pl.when(sid == 0)
        def _():
            pltpu.sync_copy(zd_hbm, den_sp)

        plsc.subcore_barrier()

        base0 = wid * epw

        def cbase(i):
            return base0 + jnp.minimum(i, nchunks - 1) * kc

        def issue_a(i, b):
            pltpu.async_copy(src_hbm.at[pl.ds(cbase(i), kc)], src_v.at[b],
                             semA[b])
            pltpu.async_copy(dst_hbm.at[pl.ds(cbase(i), kc)], dst_v.at[b],
                             semA[b])

        def wait_a(b):
            pltpu.make_async_copy(src_hbm.at[pl.ds(0, kc)], src_v.at[b],
                                  semA[b]).wait()
            pltpu.make_async_copy(dst_hbm.at[pl.ds(0, kc)], dst_v.at[b],
                                  semA[b]).wait()

        def issue_g(b):
            pltpu.async_copy(xl_hbm.at[src_v.at[b]], xlr.at[b], semG[b])
            pltpu.async_copy(xr_hbm.at[dst_v.at[b]], xrr.at[b], semG[b])

        def wait_g(b):
            pltpu.make_async_copy(xl_hbm.at[src_v.at[b]], xlr.at[b],
                                  semG[b]).wait()
            pltpu.make_async_copy(xr_hbm.at[dst_v.at[b]], xrr.at[b],
                                  semG[b]).wait()

        def copy_dst(b):
            for grp in range(kc // LANES):
                sl = pl.ds(grp * LANES, LANES)
                dsto_v[b, sl] = dst_v[b, sl]

        def compute(i, b):
            base = cbase(i)
            for grp in range(kc // LANES):
                erow = lax.iota(jnp.int32, LANES) + grp * LANES
                inb = (erow + base) < (E + N)
                for h in range(HEADS):
                    hrow = jnp.full((LANES,), h, jnp.int32)
                    s0 = jnp.zeros((LANES,), jnp.float32)

                    @pl.loop(0, c, init_carry=s0, unroll=4)
                    def _score(cc, s):
                        colv = jnp.full((LANES,), h * c, jnp.int32) + cc
                        a = plsc.load_gather(xlr.at[b], [erow, colv])
                        bb = plsc.load_gather(xrr.at[b], [erow, colv])
                        t = a + bb
                        t = jnp.where(t > 0, t, 0.2 * t)
                        av = plsc.load_gather(
                            att_v, [hrow, jnp.full((LANES,), cc, jnp.int32)])
                        return s + av * t

                    w = jnp.where(inb, jnp.exp(_score), 0.0)
                    plsc.store_scatter(wbuf.at[b], [erow, hrow], w)

        def issue_o(i, b):
            pltpu.async_copy(wbuf.at[b], w_hbm.at[pl.ds(cbase(i), kc)],
                             semO[b])
            pltpu.sync_copy(wbuf.at[b], den_sp.at[dsto_v.at[b]], add=True)

        def wait_o(b):
            pltpu.make_async_copy(wbuf.at[b], w_hbm.at[pl.ds(0, kc)],
                                  semO[b]).wait()

        issue_a(0, 0)
        wait_a(0)
        issue_g(0)
        issue_a(1, 1)

        @pl.loop(0, npairs)
        def _pair(j):
            i0 = 2 * j
            wait_a(1)
            wait_g(0)
            issue_g(1)
            @pl.when(i0 >= 2)
            def _():
                wait_o(0)
            copy_dst(0)
            issue_a(i0 + 2, 0)
            compute(i0, 0)
            issue_o(i0, 0)
            wait_a(0)
            wait_g(1)
            issue_g(0)
            @pl.when(i0 >= 1)
            def _():
                wait_o(1)
            copy_dst(1)
            issue_a(i0 + 3, 1)
            compute(i0 + 1, 1)
            issue_o(i0 + 1, 1)

        wait_a(1)
        wait_g(0)
        if tail:
            wait_o(0)
            copy_dst(0)
            compute(2 * npairs, 0)
            issue_o(2 * npairs, 0)
            wait_o(0)
            wait_o(1)
        else:
            wait_o(0)
            wait_o(1)

        plsc.subcore_barrier()

        @pl.when(sid == 0)
        def _():
            pltpu.sync_copy(den_sp, den_hbm.at[cid])

    return score_kernel


def _make_message_kernel(d, c, kc):
    hpc = HEADS // NC
    epw = EP // NS
    nchunks = epw // kc
    npairs = nchunks // 2
    tail = nchunks - 2 * npairs

    @functools.partial(
        pl.kernel,
        out_type=jax.ShapeDtypeStruct((HEADS, N, c), jnp.float32),
        mesh=_SC_MESH,
        scratch_types=[
            pltpu.VMEM((2, kc), jnp.int32),
            pltpu.VMEM((2, kc), jnp.int32),
            pltpu.VMEM((2, kc), jnp.int32),
            pltpu.VMEM((2, kc), jnp.int32),
            pltpu.VMEM((2, kc, c), jnp.float32),
            pltpu.VMEM((2, kc, HEADS), jnp.float32),
            pltpu.VMEM_SHARED((N, c), jnp.float32),
            pltpu.SemaphoreType.DMA,
            pltpu.SemaphoreType.DMA,
            pltpu.SemaphoreType.DMA,
            pltpu.SemaphoreType.DMA,
            pltpu.SemaphoreType.DMA,
            pltpu.SemaphoreType.DMA,
        ],
        compiler_params=_SC_PARAMS,
    )
    def message_kernel(src_hbm, dst_hbm, xlhc_hbm, w_hbm, zn_hbm,
                       num_hbm,
                       src_v, dst_v, dsto_v, ridx_v, rows, wall, num_sp,
                       semA0, semA1, semG0, semG1, semO0, semO1):
        cid = lax.axis_index("c")
        sid = lax.axis_index("s")
        base0 = sid * epw
        semA = [semA0, semA1]
        semG = [semG0, semG1]
        semO = [semO0, semO1]

        def cbase(i):
            return base0 + jnp.minimum(i, nchunks - 1) * kc

        for p in range(hpc):
            hglob_s = cid * hpc + p
            pltpu.sync_copy(zn_hbm.at[pl.ds(sid * NPT, NPT)],
                            num_sp.at[pl.ds(sid * NPT, NPT)])
            plsc.subcore_barrier()

            def issue_a(i, b):
                pltpu.async_copy(src_hbm.at[pl.ds(cbase(i), kc)],
                                 src_v.at[b], semA[b])
                pltpu.async_copy(dst_hbm.at[pl.ds(cbase(i), kc)],
                                 dst_v.at[b], semA[b])
                pltpu.async_copy(w_hbm.at[pl.ds(cbase(i), kc)],
                                 wall.at[b], semA[b])

            def wait_a(b):
                pltpu.make_async_copy(src_hbm.at[pl.ds(0, kc)],
                                      src_v.at[b], semA[b]).wait()
                pltpu.make_async_copy(dst_hbm.at[pl.ds(0, kc)],
                                      dst_v.at[b], semA[b]).wait()
                pltpu.make_async_copy(w_hbm.at[pl.ds(0, kc)],
                                      wall.at[b], semA[b]).wait()

            def build_ridx(b):
                for grp in range(kc // LANES):
                    sl = pl.ds(grp * LANES, LANES)
                    ridx_v[b, sl] = src_v[b, sl] * HEADS + hglob_s

            def issue_g(b):
                pltpu.async_copy(xlhc_hbm.at[ridx_v.at[b]], rows.at[b],
                                 semG[b])

            def wait_g(b):
                pltpu.make_async_copy(xlhc_hbm.at[ridx_v.at[b]], rows.at[b],
                                      semG[b]).wait()

            def copy_dst(b):
                for grp in range(kc // LANES):
                    sl = pl.ds(grp * LANES, LANES)
                    dsto_v[b, sl] = dst_v[b, sl]

            def compute(b):
                for grp in range(kc // LANES):
                    erow = lax.iota(jnp.int32, LANES) + grp * LANES
                    hrow = jnp.full((LANES,), 0, jnp.int32) + hglob_s
                    wv = plsc.load_gather(wall.at[b], [erow, hrow])

                    @pl.loop(0, c, unroll=4)
                    def _scale(cc):
                        colv = jnp.full((LANES,), 0, jnp.int32) + cc
                        m = plsc.load_gather(rows.at[b], [erow, colv])
                        plsc.store_scatter(rows.at[b], [erow, colv], m * wv)

            def issue_o(b):
                pltpu.sync_copy(rows.at[b], num_sp.at[dsto_v.at[b]],
                                add=True)

            def wait_o(b):
                pass

            issue_a(0, 0)
            wait_a(0)
            build_ridx(0)
            issue_g(0)
            issue_a(1, 1)

            @pl.loop(0, npairs)
            def _pair(j):
                i0 = 2 * j
                wait_a(1)
                wait_g(0)
                build_ridx(1)
                issue_g(1)
                @pl.when(i0 >= 2)
                def _():
                    wait_o(0)
                copy_dst(0)
                issue_a(i0 + 2, 0)
                compute(0)
                issue_o(0)
                wait_a(0)
                wait_g(1)
                build_ridx(0)
                issue_g(0)
                @pl.when(i0 >= 1)
                def _():
                    wait_o(1)
                copy_dst(1)
                issue_a(i0 + 3, 1)
                compute(1)
                issue_o(1)

            wait_a(1)
            wait_g(0)
            if tail:
                wait_o(0)
                copy_dst(0)
                compute(0)
                issue_o(0)
            wait_o(0)
            wait_o(1)

            plsc.subcore_barrier()
            pltpu.sync_copy(num_sp.at[pl.ds(sid * NPT, NPT)],
                            num_hbm.at[hglob_s, pl.ds(sid * NPT, NPT)])
            plsc.subcore_barrier()

    return message_kernel


def _combine_mm_body(num_ref, den_ref, bias_ref, *args, c, relu, n_out):
    w_refs = args[:n_out]
    b_refs = args[n_out:2 * n_out]
    o_refs = args[2 * n_out:]
    den = den_ref[0] + den_ref[1]
    accs = [jnp.zeros(o.shape, jnp.float32) for o in o_refs]
    for h in range(HEADS):
        seg = (num_ref[h] / (den[:, h:h + 1] + 1e-16)
               + bias_ref[0, h * c:(h + 1) * c])
        if relu:
            seg = jnp.maximum(seg, 0.0)
        for j, w_ref in enumerate(w_refs):
            accs[j] += jnp.dot(seg, w_ref[h * c:(h + 1) * c, :],
                               preferred_element_type=jnp.float32)
    for j, o_ref in enumerate(o_refs):
        o_ref[...] = accs[j] + b_refs[j][...]


def _combine_mm(num, den, bias, ws, bs, *, relu, block_rows=1000):
    c = num.shape[2]
    n_out = len(ws)
    grid = (N // block_rows,)
    body = functools.partial(_combine_mm_body, c=c, relu=relu, n_out=n_out)
    in_specs = [
        pl.BlockSpec((HEADS, block_rows, c), lambda i: (0, i, 0)),
        pl.BlockSpec((NC, block_rows, HEADS), lambda i: (0, i, 0)),
        pl.BlockSpec((1, HEADS * c), lambda i: (0, 0)),
    ]
    for w in ws:
        in_specs.append(pl.BlockSpec(w.shape, lambda i: (0, 0)))
    for b in bs:
        in_specs.append(pl.BlockSpec(b.shape, lambda i: (0,)))
    outs = pl.pallas_call(
        body,
        grid=grid,
        in_specs=in_specs,
        out_specs=[pl.BlockSpec((block_rows, w.shape[1]), lambda i: (i, 0))
                   for w in ws],
        out_shape=[jax.ShapeDtypeStruct((N, w.shape[1]), jnp.float32)
                   for w in ws],
    )(num, den, bias.reshape(1, -1), *ws, *bs)
    return outs


def kernel(x, edge_index, Wl1, bl1, Wr1, br1, att1, bias1,
           Wl2, bl2, Wr2, br2, att2, bias2, Wlin, blin):
    loops = jnp.arange(N, dtype=edge_index.dtype)
    pad = jnp.zeros((EP - E - N,), dtype=edge_index.dtype)
    src = jnp.concatenate([edge_index[0], loops, pad])
    dst = jnp.concatenate([edge_index[1], loops, pad])

    xl1, xr1 = _proj_pair(x, Wl1, bl1, Wr1, br1)
    zd = jnp.zeros((N, HEADS), jnp.float32)
    w1, den1 = _make_score_kernel(512, 64, 32)(src, dst, xl1, xr1, att1, zd)
    zn1 = jnp.zeros((N, 64), jnp.float32)
    num1 = _make_message_kernel(512, 64, 64)(
        src, dst, xl1.reshape(N * HEADS, 64), w1, zn1)

    xl2, xr2 = _combine_mm(num1, den1, bias1, [Wl2, Wr2], [bl2, br2],
                           relu=True)

    w2, den2 = _make_score_kernel(1024, 128, 16)(src, dst, xl2, xr2, att2, zd)
    zn2 = jnp.zeros((N, 128), jnp.float32)
    num2 = _make_message_kernel(1024, 128, 64)(
        src, dst, xl2.reshape(N * HEADS, 128), w2, zn2)

    (out,) = _combine_mm(num2, den2, bias2, [Wlin], [blin], relu=False)
    return out


# speedup vs baseline: 15.3215x; 6.3135x over previous
import functools

import jax
import jax.numpy as jnp
from jax import lax
from jax.experimental import pallas as pl
from jax.experimental.pallas import tpu as pltpu
from jax.experimental.pallas import tpu_sc as plsc

N = 10000
E = 160000
HEADS = 8
NC = 2
NS = 16
NW = NC * NS
LANES = 16
KC = 32
EP = 167 * NW * KC
NPT = N // NS

_SC_MESH = plsc.VectorSubcoreMesh(core_axis_name="c", subcore_axis_name="s")
_SC_PARAMS = pltpu.CompilerParams(use_tc_tiling_on_sc=False,
                                  needs_layout_passes=False)


def _mm2_body(x_ref, wl_ref, bl_ref, wr_ref, br_ref, ol_ref, or_ref):
    x = x_ref[...]
    ol_ref[...] = (
        jnp.dot(x, wl_ref[...], preferred_element_type=jnp.float32) + bl_ref[...]
    )
    or_ref[...] = (
        jnp.dot(x, wr_ref[...], preferred_element_type=jnp.float32) + br_ref[...]
    )


def _proj_pair(x, wl, bl, wr, br, block_rows=1000):
    n, k = x.shape
    m = wl.shape[1]
    grid = (n // block_rows,)
    return pl.pallas_call(
        _mm2_body,
        grid=grid,
        in_specs=[
            pl.BlockSpec((block_rows, k), lambda i: (i, 0)),
            pl.BlockSpec((k, m), lambda i: (0, 0)),
            pl.BlockSpec((m,), lambda i: (0,)),
            pl.BlockSpec((k, m), lambda i: (0, 0)),
            pl.BlockSpec((m,), lambda i: (0,)),
        ],
        out_specs=[
            pl.BlockSpec((block_rows, m), lambda i: (i, 0)),
            pl.BlockSpec((block_rows, m), lambda i: (i, 0)),
        ],
        out_shape=[
            jax.ShapeDtypeStruct((n, m), jnp.float32),
            jax.ShapeDtypeStruct((n, m), jnp.float32),
        ],
    )(x, wl, bl, wr, br)


def _make_score_kernel(d, c, kc):
    epw = EP // NW
    nchunks = epw // kc
    npairs = nchunks // 2
    tail = nchunks - 2 * npairs

    @functools.partial(
        pl.kernel,
        out_type=[
            jax.ShapeDtypeStruct((EP, HEADS), jnp.float32),
            jax.ShapeDtypeStruct((NC, N, HEADS), jnp.float32),
        ],
        mesh=_SC_MESH,
        scratch_types=[
            pltpu.VMEM((2, kc), jnp.int32),
            pltpu.VMEM((2, kc), jnp.int32),
            pltpu.VMEM((2, kc), jnp.int32),
            pltpu.VMEM((2, kc, d), jnp.float32),
            pltpu.VMEM((2, kc, d), jnp.float32),
            pltpu.VMEM((2, kc, HEADS), jnp.float32),
            pltpu.VMEM((HEADS, c), jnp.float32),
            pltpu.VMEM_SHARED((N, HEADS), jnp.float32),
            pltpu.SemaphoreType.DMA,
            pltpu.SemaphoreType.DMA,
            pltpu.SemaphoreType.DMA,
            pltpu.SemaphoreType.DMA,
            pltpu.SemaphoreType.DMA,
            pltpu.SemaphoreType.DMA,
        ],
        compiler_params=_SC_PARAMS,
    )
    def score_kernel(src_hbm, dst_hbm, xl_hbm, xr_hbm, att_hbm, zd_hbm,
                     w_hbm, den_hbm,
                     src_v, dst_v, dsto_v, xlr, xrr, wbuf, att_v, den_sp,
                     semA0, semA1, semG0, semG1, semO0, semO1):
        cid = lax.axis_index("c")
        sid = lax.axis_index("s")
        wid = sid * NC + cid
        semA = [semA0, semA1]
        semG = [semG0, semG1]
        semO = [semO0, semO1]

        pltpu.sync_copy(att_hbm, att_v)

        @pl.when(sid == 0)
        def _():
            pltpu.sync_copy(zd_hbm, den_sp)

        plsc.subcore_barrier()

        base0 = wid * epw

        def cbase(i):
            return base0 + jnp.minimum(i, nchunks - 1) * kc

        def issue_a(i, b):
            pltpu.async_copy(src_hbm.at[pl.ds(cbase(i), kc)], src_v.at[b],
                             semA[b])
            pltpu.async_copy(dst_hbm.at[pl.ds(cbase(i), kc)], dst_v.at[b],
                             semA[b])

        def wait_a(b):
            pltpu.make_async_copy(src_hbm.at[pl.ds(0, kc)], src_v.at[b],
                                  semA[b]).wait()
            pltpu.make_async_copy(dst_hbm.at[pl.ds(0, kc)], dst_v.at[b],
                                  semA[b]).wait()

        def issue_g(b):
            pltpu.async_copy(xl_hbm.at[src_v.at[b]], xlr.at[b], semG[b])
            pltpu.async_copy(xr_hbm.at[dst_v.at[b]], xrr.at[b], semG[b])

        def wait_g(b):
            pltpu.make_async_copy(xl_hbm.at[src_v.at[b]], xlr.at[b],
                                  semG[b]).wait()
            pltpu.make_async_copy(xr_hbm.at[dst_v.at[b]], xrr.at[b],
                                  semG[b]).wait()

        def copy_dst(b):
            for grp in range(kc // LANES):
                sl = pl.ds(grp * LANES, LANES)
                dsto_v[b, sl] = dst_v[b, sl]

        def compute(i, b):
            base = cbase(i)
            lane = lax.iota(jnp.int32, LANES)
            hidx = lax.bitwise_and(lane, 7)

            @pl.loop(0, kc // 2)
            def _epair(ep):
                e0 = ep * 2
                svec = jnp.zeros((LANES,), jnp.float32)
                for k in range(2):
                    for h in range(HEADS):
                        acc = jnp.zeros((LANES,), jnp.float32)
                        for g in range(c // LANES):
                            sl = pl.ds(h * c + g * LANES, LANES)
                            a = xlr[b, e0 + k, sl]
                            r = xrr[b, e0 + k, sl]
                            t = a + r
                            t = jnp.where(t > 0, t, 0.2 * t)
                            acc = acc + t * att_v[h, pl.ds(g * LANES, LANES)]
                        svec = jnp.where(lane == k * HEADS + h,
                                         jnp.sum(acc), svec)
                eidx = e0 + lax.shift_right_logical(lane, 3)
                inb = (eidx + base) < (E + N)
                w = jnp.where(inb, jnp.exp(svec), 0.0)
                plsc.store_scatter(wbuf.at[b], [eidx, hidx], w)

        def issue_o(i, b):
            pltpu.async_copy(wbuf.at[b], w_hbm.at[pl.ds(cbase(i), kc)],
                             semO[b])
            pltpu.sync_copy(wbuf.at[b], den_sp.at[dsto_v.at[b]], add=True)

        def wait_o(b):
            pltpu.make_async_copy(wbuf.at[b], w_hbm.at[pl.ds(0, kc)],
                                  semO[b]).wait()

        issue_a(0, 0)
        wait_a(0)
        issue_g(0)
        issue_a(1, 1)

        @pl.loop(0, npairs)
        def _pair(j):
            i0 = 2 * j
            wait_a(1)
            wait_g(0)
            issue_g(1)
            @pl.when(i0 >= 2)
            def _():
                wait_o(0)
            copy_dst(0)
            issue_a(i0 + 2, 0)
            compute(i0, 0)
            issue_o(i0, 0)
            wait_a(0)
            wait_g(1)
            issue_g(0)
            @pl.when(i0 >= 1)
            def _():
                wait_o(1)
            copy_dst(1)
            issue_a(i0 + 3, 1)
            compute(i0 + 1, 1)
            issue_o(i0 + 1, 1)

        wait_a(1)
        wait_g(0)
        if tail:
            wait_o(0)
            copy_dst(0)
            compute(2 * npairs, 0)
            issue_o(2 * npairs, 0)
            wait_o(0)
            wait_o(1)
        else:
            wait_o(0)
            wait_o(1)

        plsc.subcore_barrier()

        @pl.when(sid == 0)
        def _():
            pltpu.sync_copy(den_sp, den_hbm.at[cid])

    return score_kernel


def _make_message_kernel(d, c, kc):
    hpc = HEADS // NC
    epw = EP // NS
    nchunks = epw // kc
    npairs = nchunks // 2
    tail = nchunks - 2 * npairs

    @functools.partial(
        pl.kernel,
        out_type=jax.ShapeDtypeStruct((HEADS, N, c), jnp.float32),
        mesh=_SC_MESH,
        scratch_types=[
            pltpu.VMEM((2, kc), jnp.int32),
            pltpu.VMEM((2, kc), jnp.int32),
            pltpu.VMEM((2, kc), jnp.int32),
            pltpu.VMEM((2, kc), jnp.int32),
            pltpu.VMEM((2, kc, c), jnp.float32),
            pltpu.VMEM((2, kc, HEADS), jnp.float32),
            pltpu.VMEM_SHARED((N, c), jnp.float32),
            pltpu.SemaphoreType.DMA,
            pltpu.SemaphoreType.DMA,
            pltpu.SemaphoreType.DMA,
            pltpu.SemaphoreType.DMA,
            pltpu.SemaphoreType.DMA,
            pltpu.SemaphoreType.DMA,
        ],
        compiler_params=_SC_PARAMS,
    )
    def message_kernel(src_hbm, dst_hbm, xlhc_hbm, w_hbm, zn_hbm,
                       num_hbm,
                       src_v, dst_v, dsto_v, ridx_v, rows, wall, num_sp,
                       semA0, semA1, semG0, semG1, semO0, semO1):
        cid = lax.axis_index("c")
        sid = lax.axis_index("s")
        base0 = sid * epw
        semA = [semA0, semA1]
        semG = [semG0, semG1]
        semO = [semO0, semO1]

        def cbase(i):
            return base0 + jnp.minimum(i, nchunks - 1) * kc

        for p in range(hpc):
            hglob_s = cid * hpc + p
            pltpu.sync_copy(zn_hbm.at[pl.ds(sid * NPT, NPT)],
                            num_sp.at[pl.ds(sid * NPT, NPT)])
            plsc.subcore_barrier()

            def issue_a(i, b):
                pltpu.async_copy(src_hbm.at[pl.ds(cbase(i), kc)],
                                 src_v.at[b], semA[b])
                pltpu.async_copy(dst_hbm.at[pl.ds(cbase(i), kc)],
                                 dst_v.at[b], semA[b])
                pltpu.async_copy(w_hbm.at[pl.ds(cbase(i), kc)],
                                 wall.at[b], semA[b])

            def wait_a(b):
                pltpu.make_async_copy(src_hbm.at[pl.ds(0, kc)],
                                      src_v.at[b], semA[b]).wait()
                pltpu.make_async_copy(dst_hbm.at[pl.ds(0, kc)],
                                      dst_v.at[b], semA[b]).wait()
                pltpu.make_async_copy(w_hbm.at[pl.ds(0, kc)],
                                      wall.at[b], semA[b]).wait()

            def build_ridx(b):
                for grp in range(kc // LANES):
                    sl = pl.ds(grp * LANES, LANES)
                    ridx_v[b, sl] = src_v[b, sl] * HEADS + hglob_s

            def issue_g(b):
                pltpu.async_copy(xlhc_hbm.at[ridx_v.at[b]], rows.at[b],
                                 semG[b])

            def wait_g(b):
                pltpu.make_async_copy(xlhc_hbm.at[ridx_v.at[b]], rows.at[b],
                                      semG[b]).wait()

            def copy_dst(b):
                for grp in range(kc // LANES):
                    sl = pl.ds(grp * LANES, LANES)
                    dsto_v[b, sl] = dst_v[b, sl]

            def compute(b):
                @pl.loop(0, kc, unroll=4)
                def _edge(e):
                    wv = plsc.load_gather(
                        wall.at[b],
                        [jnp.full((LANES,), 0, jnp.int32) + e,
                         jnp.full((LANES,), hglob_s, jnp.int32)])
                    for g in range(c // LANES):
                        sl = pl.ds(g * LANES, LANES)
                        rows[b, e, sl] = rows[b, e, sl] * wv

            def issue_o(b):
                pltpu.sync_copy(rows.at[b], num_sp.at[dsto_v.at[b]],
                                add=True)

            def wait_o(b):
                pass

            issue_a(0, 0)
            wait_a(0)
            build_ridx(0)
            issue_g(0)
            issue_a(1, 1)

            @pl.loop(0, npairs)
            def _pair(j):
                i0 = 2 * j
                wait_a(1)
                wait_g(0)
                build_ridx(1)
                issue_g(1)
                @pl.when(i0 >= 2)
                def _():
                    wait_o(0)
                copy_dst(0)
                issue_a(i0 + 2, 0)
                compute(0)
                issue_o(0)
                wait_a(0)
                wait_g(1)
                build_ridx(0)
                issue_g(0)
                @pl.when(i0 >= 1)
                def _():
                    wait_o(1)
                copy_dst(1)
                issue_a(i0 + 3, 1)
                compute(1)
                issue_o(1)

            wait_a(1)
            wait_g(0)
            if tail:
                wait_o(0)
                copy_dst(0)
                compute(0)
                issue_o(0)
            wait_o(0)
            wait_o(1)

            plsc.subcore_barrier()
            pltpu.sync_copy(num_sp.at[pl.ds(sid * NPT, NPT)],
                            num_hbm.at[hglob_s, pl.ds(sid * NPT, NPT)])
            plsc.subcore_barrier()

    return message_kernel


def _combine_mm_body(num_ref, den_ref, bias_ref, *args, c, relu, n_out):
    w_refs = args[:n_out]
    b_refs = args[n_out:2 * n_out]
    o_refs = args[2 * n_out:]
    den = den_ref[0] + den_ref[1]
    accs = [jnp.zeros(o.shape, jnp.float32) for o in o_refs]
    for h in range(HEADS):
        seg = (num_ref[h] / (den[:, h:h + 1] + 1e-16)
               + bias_ref[0, h * c:(h + 1) * c])
        if relu:
            seg = jnp.maximum(seg, 0.0)
        for j, w_ref in enumerate(w_refs):
            accs[j] += jnp.dot(seg, w_ref[h * c:(h + 1) * c, :],
                               preferred_element_type=jnp.float32)
    for j, o_ref in enumerate(o_refs):
        o_ref[...] = accs[j] + b_refs[j][...]


def _combine_mm(num, den, bias, ws, bs, *, relu, block_rows=1000):
    c = num.shape[2]
    n_out = len(ws)
    grid = (N // block_rows,)
    body = functools.partial(_combine_mm_body, c=c, relu=relu, n_out=n_out)
    in_specs = [
        pl.BlockSpec((HEADS, block_rows, c), lambda i: (0, i, 0)),
        pl.BlockSpec((NC, block_rows, HEADS), lambda i: (0, i, 0)),
        pl.BlockSpec((1, HEADS * c), lambda i: (0, 0)),
    ]
    for w in ws:
        in_specs.append(pl.BlockSpec(w.shape, lambda i: (0, 0)))
    for b in bs:
        in_specs.append(pl.BlockSpec(b.shape, lambda i: (0,)))
    outs = pl.pallas_call(
        body,
        grid=grid,
        in_specs=in_specs,
        out_specs=[pl.BlockSpec((block_rows, w.shape[1]), lambda i: (i, 0))
                   for w in ws],
        out_shape=[jax.ShapeDtypeStruct((N, w.shape[1]), jnp.float32)
                   for w in ws],
    )(num, den, bias.reshape(1, -1), *ws, *bs)
    return outs


def kernel(x, edge_index, Wl1, bl1, Wr1, br1, att1, bias1,
           Wl2, bl2, Wr2, br2, att2, bias2, Wlin, blin):
    loops = jnp.arange(N, dtype=edge_index.dtype)
    pad = jnp.zeros((EP - E - N,), dtype=edge_index.dtype)
    src = jnp.concatenate([edge_index[0], loops, pad])
    dst = jnp.concatenate([edge_index[1], loops, pad])

    xl1, xr1 = _proj_pair(x, Wl1, bl1, Wr1, br1)
    zd = jnp.zeros((N, HEADS), jnp.float32)
    w1, den1 = _make_score_kernel(512, 64, 32)(src, dst, xl1, xr1, att1, zd)
    zn1 = jnp.zeros((N, 64), jnp.float32)
    num1 = _make_message_kernel(512, 64, 64)(
        src, dst, xl1.reshape(N * HEADS, 64), w1, zn1)

    xl2, xr2 = _combine_mm(num1, den1, bias1, [Wl2, Wr2], [bl2, br2],
                           relu=True)

    w2, den2 = _make_score_kernel(1024, 128, 16)(src, dst, xl2, xr2, att2, zd)
    zn2 = jnp.zeros((N, 128), jnp.float32)
    num2 = _make_message_kernel(1024, 128, 64)(
        src, dst, xl2.reshape(N * HEADS, 128), w2, zn2)

    (out,) = _combine_mm(num2, den2, bias2, [Wlin], [blin], relu=False)
    return out
